# baseline (speedup 1.0000x reference)
import functools

import jax
import jax.numpy as jnp
from jax.experimental import pallas as pl

N_IMG = 8
ROI_PER_IMG = 32
N_ROI = N_IMG * ROI_PER_IMG
C_FEAT = 128
CLASS_NUM = 80
POOL_H = 16
POOL_W = 16
HW = POOL_H * POOL_W
K_TOP = 100
NMS_THRESH = 0.5

ROIB = 4


def _heads_body(p_ref, wk_ref, b1_ref, w2h_ref, b2h_ref, wow_ref, bow_ref,
                hm_ref, ow_ref):
    x = p_ref[...].reshape(ROIB * HW, 9 * C_FEAT)
    h = jnp.dot(x, wk_ref[...], preferred_element_type=jnp.float32,
                precision=jax.lax.Precision.DEFAULT)
    h = h + b1_ref[...]
    h = jnp.maximum(h, 0.0)
    hm = jnp.dot(h[:, 0:C_FEAT], w2h_ref[...], preferred_element_type=jnp.float32,
                 precision=jax.lax.Precision.DEFAULT)
    hm = hm + b2h_ref[...]
    ow = jnp.dot(h[:, C_FEAT:3 * C_FEAT], wow_ref[...],
                 preferred_element_type=jnp.float32,
                 precision=jax.lax.Precision.DEFAULT)
    ow = ow + bow_ref[...]
    hm_ref[...] = hm.reshape(ROIB, HW, CLASS_NUM)
    ow_ref[...] = ow.reshape(ROIB, HW, 4)


def _conv_heads(features, params):
    xt = features.transpose(0, 2, 3, 1)
    xp = jnp.pad(xt, ((0, 0), (1, 1), (1, 1), (0, 0)))
    patches = jnp.concatenate(
        [xp[:, dy:dy + POOL_H, dx:dx + POOL_W, :].reshape(N_ROI, HW, C_FEAT)
         for dy in range(3) for dx in range(3)], axis=-1)

    w1_all = jnp.concatenate(
        [params['heatmap_w1'], params['offset_w1'], params['width_height_w1']],
        axis=0)
    wk = w1_all.transpose(2, 3, 1, 0).reshape(9 * C_FEAT, 3 * C_FEAT)
    b1_all = jnp.concatenate(
        [params['heatmap_b1'], params['offset_b1'], params['width_height_b1']],
        axis=0)[None, :]
    w2h = params['heatmap_w2'][:, :, 0, 0].T
    b2h = params['heatmap_b2'][None, :]
    wow = jnp.zeros((2 * C_FEAT, 4), jnp.float32)
    wow = wow.at[0:C_FEAT, 0:2].set(params['offset_w2'][:, :, 0, 0].T)
    wow = wow.at[C_FEAT:2 * C_FEAT, 2:4].set(params['width_height_w2'][:, :, 0, 0].T)
    bow = jnp.concatenate([params['offset_b2'], params['width_height_b2']])[None, :]

    grid = (N_ROI // ROIB,)
    hm, ow = pl.pallas_call(
        _heads_body,
        grid=grid,
        in_specs=[
            pl.BlockSpec((ROIB, HW, 9 * C_FEAT), lambda i: (i, 0, 0)),
            pl.BlockSpec((9 * C_FEAT, 3 * C_FEAT), lambda i: (0, 0)),
            pl.BlockSpec((1, 3 * C_FEAT), lambda i: (0, 0)),
            pl.BlockSpec((C_FEAT, CLASS_NUM), lambda i: (0, 0)),
            pl.BlockSpec((1, CLASS_NUM), lambda i: (0, 0)),
            pl.BlockSpec((2 * C_FEAT, 4), lambda i: (0, 0)),
            pl.BlockSpec((1, 4), lambda i: (0, 0)),
        ],
        out_specs=[
            pl.BlockSpec((ROIB, HW, CLASS_NUM), lambda i: (i, 0, 0)),
            pl.BlockSpec((ROIB, HW, 4), lambda i: (i, 0, 0)),
        ],
        out_shape=[
            jax.ShapeDtypeStruct((N_ROI, HW, CLASS_NUM), jnp.float32),
            jax.ShapeDtypeStruct((N_ROI, HW, 4), jnp.float32),
        ],
    )(patches, wk, b1_all, w2h, b2h, wow, bow)
    return hm, ow


def _point_nms(hm):
    hmax = jax.lax.reduce_window(hm, -jnp.inf, jax.lax.max, (1, 1, 3, 3),
                                 (1, 1, 1, 1), [(0, 0), (0, 0), (1, 1), (1, 1)])
    keep = (hmax == hm).astype(hm.dtype)
    return hm * keep


def _batched_nms(boxes, scores, labels, iou_thr):
    boxes = boxes.astype(jnp.float32)
    span = boxes.max() - boxes.min() + 1.0
    b = boxes + (labels.astype(boxes.dtype) * span)[:, None]
    x1, y1, x2, y2 = b[:, 0], b[:, 1], b[:, 2], b[:, 3]
    areas = jnp.maximum(x2 - x1, 0.0) * jnp.maximum(y2 - y1, 0.0)
    order = jnp.argsort(-scores)
    n = scores.shape[0]
    suppressed0 = jnp.zeros((n,), dtype=bool)
    keep0 = jnp.zeros((n,), dtype=bool)

    def body(t, state):
        suppressed, keep = state
        i = order[t]
        active = jnp.logical_not(suppressed[i])
        keep = keep.at[i].set(jnp.logical_or(keep[i], active))
        xx1 = jnp.maximum(x1[i], x1)
        yy1 = jnp.maximum(y1[i], y1)
        xx2 = jnp.minimum(x2[i], x2)
        yy2 = jnp.minimum(y2[i], y2)
        inter = jnp.maximum(xx2 - xx1, 0.0) * jnp.maximum(yy2 - yy1, 0.0)
        iou = inter / (areas[i] + areas - inter + 1e-9)
        newly = jnp.logical_and(iou > iou_thr, active)
        newly = newly.at[i].set(False)
        suppressed = jnp.logical_or(suppressed, newly)
        return suppressed, keep

    _, keep = jax.lax.fori_loop(0, n, body, (suppressed0, keep0))
    return keep


def kernel(roi_boxes, features, params, inputs, stride):
    hm_l, ow = _conv_heads(features, params)
    heatmap = jax.nn.sigmoid(hm_l.transpose(0, 2, 1).reshape(
        N_ROI, CLASS_NUM, POOL_H, POOL_W))
    offset = ow[:, :, 0:2].transpose(0, 2, 1)
    width_height = ow[:, :, 2:4].transpose(0, 2, 1)

    hm = _point_nms(heatmap)
    flat = hm.reshape(N_ROI, CLASS_NUM * HW)
    scores, inds_full = jax.lax.top_k(flat, K_TOP)
    categories = inds_full // HW
    spatial = inds_full % HW
    ys = (spatial // POOL_W).astype(jnp.float32)
    xs = (spatial % POOL_W).astype(jnp.float32)

    idx = jnp.broadcast_to(spatial[:, None, :], (N_ROI, 2, K_TOP))
    offset_k = jnp.take_along_axis(offset, idx, axis=2)
    wh_k = jnp.take_along_axis(width_height, idx, axis=2)

    rb = roi_boxes.reshape(-1, 4)
    roi_w = rb[:, 2] - rb[:, 0]
    roi_h = rb[:, 3] - rb[:, 1]
    w_scale = (roi_w / POOL_W)[:, None]
    h_scale = (roi_h / POOL_H)[:, None]
    xs2 = (xs + offset_k[:, 0, :]) * w_scale
    ys2 = (ys + offset_k[:, 1, :]) * h_scale
    width = wh_k[:, 0, :] * w_scale / stride
    height = wh_k[:, 1, :] * h_scale / stride
    x1 = xs2 - width / 2 + rb[:, 0][:, None]
    x2 = xs2 + width / 2 + rb[:, 0][:, None]
    y1 = ys2 - height / 2 + rb[:, 1][:, None]
    y2 = ys2 + height / 2 + rb[:, 1][:, None]
    boxes = jnp.stack([x1, y1, x2, y2], axis=2)

    labels = categories + 1
    boxes_im = boxes.reshape(N_IMG, ROI_PER_IMG * K_TOP, 4)
    scores_im = scores.reshape(N_IMG, ROI_PER_IMG * K_TOP)
    labels_im = labels.reshape(N_IMG, ROI_PER_IMG * K_TOP)
    keep_mask = jax.vmap(
        lambda b, s, l: _batched_nms(b, s, l, NMS_THRESH))(
            boxes_im, scores_im, labels_im)
    return (heatmap, offset_k, wh_k, boxes_im, scores_im, labels_im, keep_mask)
